# baseline (speedup 1.0000x reference)
import functools

import jax
import jax.numpy as jnp
from jax import lax
from jax.experimental import pallas as pl
from jax.experimental.pallas import tpu as pltpu
from jax.experimental.pallas import tpu_sc as plsc

VOCAB = 1000000
DIM = 64
WORLD_SIZE = 4
RANK = 0
PART = VOCAB // WORLD_SIZE
START = RANK * PART
END = START + PART

NUM_CORES = 2
NUM_SUBCORES = 16
NUM_WORKERS = NUM_CORES * NUM_SUBCORES
LANES = 16
VPR = DIM // LANES

CHUNK = 400
SUBC = 80
SUB = CHUNK // SUBC
IGNORED = -1


def _sc_gather(n_batch, n_hist):
    n_idx = n_batch * n_hist
    per_w = n_idx // NUM_WORKERS
    n_super = per_w // CHUNK
    n_half = n_super // 2
    nb = CHUNK // n_hist
    assert per_w % CHUNK == 0 and n_super % 2 == 0 and per_w % LANES == 0
    assert CHUNK % n_hist == 0

    mesh = plsc.VectorSubcoreMesh(core_axis_name="c", subcore_axis_name="s")

    @functools.partial(
        pl.kernel,
        out_type=jax.ShapeDtypeStruct((n_batch, n_hist, DIM), jnp.float32),
        mesh=mesh,
        scratch_types=[
            pltpu.VMEM((per_w,), jnp.int32),
            pltpu.VMEM((CHUNK, DIM), jnp.float32),
            pltpu.VMEM((CHUNK, DIM), jnp.float32),
            pltpu.VMEM((nb, n_hist, DIM), jnp.float32),
            pltpu.SemaphoreType.DMA,
            pltpu.SemaphoreType.DMA,
        ],
        compiler_params=pltpu.CompilerParams(use_tc_tiling_on_sc=False),
    )
    def k(idx_hbm, table_hbm, out3d_hbm, idx_v, rows0, rows1, stage, sem0, sem1):
        wid = lax.axis_index("s") * NUM_CORES + lax.axis_index("c")
        base = wid * per_w
        batch0 = wid * (per_w // n_hist)

        pltpu.sync_copy(idx_hbm.at[pl.ds(base, per_w)], idx_v)

        def remap(i, carry):
            v = idx_v[pl.ds(i * LANES, LANES)]
            m = (v < START) | (v >= END)
            idx_v[pl.ds(i * LANES, LANES)] = jnp.where(m, IGNORED, v - START)
            return carry

        lax.fori_loop(0, per_w // LANES, remap, 0, unroll=4)

        zeros = jnp.zeros((LANES,), jnp.float32)

        def zero_buf(buf):
            def z(j, carry):
                buf[j // VPR, pl.ds((j % VPR) * LANES, LANES)] = zeros
                return carry

            lax.fori_loop(0, CHUNK * VPR, z, 0, unroll=8)

        def gather_src(c, s):
            return table_hbm.at[
                plsc.Indices(
                    idx_v.at[pl.ds(c * CHUNK + s * SUBC, SUBC)],
                    ignored_value=IGNORED,
                )
            ]

        def start_gathers(c, buf, sem):
            for s in range(SUB):
                pltpu.async_copy(
                    gather_src(c, s), buf.at[pl.ds(s * SUBC, SUBC)], sem
                )

        def wait_gathers(c, buf, sem):
            for s in range(SUB):
                pltpu.make_async_copy(
                    gather_src(c, s), buf.at[pl.ds(s * SUBC, SUBC)], sem
                ).wait()

        def shuffle_write(c, buf):
            for b in range(nb):
                def mv(h, carry):
                    for q in range(VPR):
                        stage[b, h, pl.ds(q * LANES, LANES)] = buf[
                            b * n_hist + h, pl.ds(q * LANES, LANES)
                        ]
                    return carry

                lax.fori_loop(0, n_hist, mv, 0, unroll=2)
            pltpu.sync_copy(stage, out3d_hbm.at[pl.ds(batch0 + c * nb, nb)])

        zero_buf(rows0)
        zero_buf(rows1)
        start_gathers(0, rows0, sem0)

        def body(i, carry):
            c0 = 2 * i
            start_gathers(c0 + 1, rows1, sem1)
            wait_gathers(c0, rows0, sem0)
            shuffle_write(c0, rows0)

            @pl.when(i < n_half - 1)
            def _():
                zero_buf(rows0)
                start_gathers(c0 + 2, rows0, sem0)

            wait_gathers(c0 + 1, rows1, sem1)
            shuffle_write(c0 + 1, rows1)

            @pl.when(i < n_half - 1)
            def _():
                zero_buf(rows1)

            return carry

        lax.fori_loop(0, n_half, body, 0)

    return k


def kernel(x, weight):
    n_idx = x.shape[0] * x.shape[1]
    x_flat = x.reshape(n_idx).astype(jnp.int32)
    return _sc_gather(x.shape[0], x.shape[1])(x_flat, weight)


# speedup vs baseline: 1.3583x; 1.3583x over previous
import functools

import jax
import jax.numpy as jnp
from jax import lax
from jax.experimental import pallas as pl
from jax.experimental.pallas import tpu as pltpu
from jax.experimental.pallas import tpu_sc as plsc

VOCAB = 1000000
DIM = 64
WORLD_SIZE = 4
RANK = 0
PART = VOCAB // WORLD_SIZE
START = RANK * PART
END = START + PART

NUM_CORES = 2
NUM_SUBCORES = 16
NUM_WORKERS = NUM_CORES * NUM_SUBCORES
LANES = 16

CHUNK = 640
SUBC = 128
SUB = CHUNK // SUBC
IGNORED = -1


def _sc_gather(n_idx):
    per_w = n_idx // NUM_WORKERS
    n_super = per_w // CHUNK
    n_half = n_super // 2
    assert per_w % CHUNK == 0 and n_super % 2 == 0 and per_w % LANES == 0

    mesh = plsc.VectorSubcoreMesh(core_axis_name="c", subcore_axis_name="s")

    @functools.partial(
        pl.kernel,
        out_type=jax.ShapeDtypeStruct((n_idx, DIM), jnp.float32),
        mesh=mesh,
        scratch_types=[
            pltpu.VMEM((per_w,), jnp.int32),
            pltpu.VMEM((CHUNK, DIM), jnp.float32),
            pltpu.VMEM((CHUNK, DIM), jnp.float32),
            pltpu.SemaphoreType.DMA,
            pltpu.SemaphoreType.DMA,
        ],
        compiler_params=pltpu.CompilerParams(use_tc_tiling_on_sc=False),
    )
    def k(idx_hbm, table_hbm, out_hbm, idx_v, rows0, rows1, sem0, sem1):
        wid = lax.axis_index("s") * NUM_CORES + lax.axis_index("c")
        base = wid * per_w

        pltpu.sync_copy(idx_hbm.at[pl.ds(base, per_w)], idx_v)

        def remap(i, carry):
            v = idx_v[pl.ds(i * LANES, LANES)]
            m = (v < START) | (v >= END)
            idx_v[pl.ds(i * LANES, LANES)] = jnp.where(m, IGNORED, v - START)
            return carry

        lax.fori_loop(0, per_w // LANES, remap, 0, unroll=4)

        zeros = jnp.zeros((LANES,), jnp.float32)

        def zero_buf(buf):
            def z(j, carry):
                buf[j // (DIM // LANES), pl.ds((j % (DIM // LANES)) * LANES, LANES)] = zeros
                return carry

            lax.fori_loop(0, CHUNK * DIM // LANES, z, 0, unroll=8)

        def gather_src(c, s):
            return table_hbm.at[
                plsc.Indices(
                    idx_v.at[pl.ds(c * CHUNK + s * SUBC, SUBC)],
                    ignored_value=IGNORED,
                )
            ]

        def start_gathers(c, buf, sem):
            for s in range(SUB):
                pltpu.async_copy(
                    gather_src(c, s), buf.at[pl.ds(s * SUBC, SUBC)], sem
                )

        def wait_gathers(c, buf, sem):
            for s in range(SUB):
                pltpu.make_async_copy(
                    gather_src(c, s), buf.at[pl.ds(s * SUBC, SUBC)], sem
                ).wait()

        def write_out(c, buf):
            pltpu.sync_copy(buf, out_hbm.at[pl.ds(base + c * CHUNK, CHUNK)])

        zero_buf(rows0)
        zero_buf(rows1)
        start_gathers(0, rows0, sem0)

        def body(i, carry):
            c0 = 2 * i
            start_gathers(c0 + 1, rows1, sem1)
            wait_gathers(c0, rows0, sem0)
            write_out(c0, rows0)

            @pl.when(i < n_half - 1)
            def _():
                zero_buf(rows0)
                start_gathers(c0 + 2, rows0, sem0)

            wait_gathers(c0 + 1, rows1, sem1)
            write_out(c0 + 1, rows1)

            @pl.when(i < n_half - 1)
            def _():
                zero_buf(rows1)

            return carry

        lax.fori_loop(0, n_half, body, 0)

    return k


def kernel(x, weight):
    n_idx = x.shape[0] * x.shape[1]
    x_flat = x.reshape(n_idx).astype(jnp.int32)
    out = _sc_gather(n_idx)(x_flat, weight)
    return out.reshape(x.shape[0], x.shape[1], DIM)
